# Initial kernel scaffold; baseline (speedup 1.0000x reference)
import jax
import jax.numpy as jnp
from jax.experimental import pallas as pl


def kernel(x, edge_index, edge_attr, batch, pos, params):
    raise NotImplementedError("write your pallas kernel here")


# speedup vs baseline: 2.8014x; 2.8014x over previous
import functools

import jax
import jax.numpy as jnp
from jax import lax
from jax.experimental import pallas as pl
from jax.experimental.pallas import tpu as pltpu
from jax.experimental.pallas import tpu_sc as plsc

N_NODES = 10000
N_EDGES = 160000
D = 256
G = 8
CUTOFF = 10.0
EPS = 1e-5

NC = 2
NS = 16
NW = NC * NS

NODE_BLK = 1000
EDGE_BLK = 2000
NPAD = 10112
RPT = NPAD // NS

@functools.cache
def _sc_mesh():
    return plsc.VectorSubcoreMesh(core_axis_name="c", subcore_axis_name="s",
                                  num_cores=NC, num_subcores=NS)


EPADT = 5008
EPAD = EPADT * NW


def _sc_el2(px, py, pz, src_pad, dst_pad):
    @functools.partial(
        pl.kernel,
        out_type=jax.ShapeDtypeStruct((EPAD,), jnp.float32),
        mesh=_sc_mesh(),
        compiler_params=pltpu.CompilerParams(needs_layout_passes=False),
        scratch_types=[
            pltpu.VMEM((N_NODES,), jnp.float32),
            pltpu.VMEM((N_NODES,), jnp.float32),
            pltpu.VMEM((N_NODES,), jnp.float32),
            pltpu.VMEM((EPADT,), jnp.int32),
            pltpu.VMEM((EPADT,), jnp.int32),
            pltpu.VMEM((EPADT,), jnp.float32),
        ],
    )
    def ek(px_hbm, py_hbm, pz_hbm, src_hbm, dst_hbm, out_hbm,
           px_v, py_v, pz_v, src_v, dst_v, el2_v):
        wid = lax.axis_index("s") * NC + lax.axis_index("c")
        base = wid * EPADT
        pltpu.sync_copy(px_hbm, px_v)
        pltpu.sync_copy(py_hbm, py_v)
        pltpu.sync_copy(pz_hbm, pz_v)
        pltpu.sync_copy(src_hbm.at[pl.ds(base, EPADT)], src_v)
        pltpu.sync_copy(dst_hbm.at[pl.ds(base, EPADT)], dst_v)

        @pl.loop(0, EPADT // 16)
        def _(j):
            off = j * 16
            s16 = src_v[pl.ds(off, 16)]
            d16 = dst_v[pl.ds(off, 16)]
            dx = plsc.load_gather(px_v, [d16]) - plsc.load_gather(px_v, [s16])
            dy = plsc.load_gather(py_v, [d16]) - plsc.load_gather(py_v, [s16])
            dz = plsc.load_gather(pz_v, [d16]) - plsc.load_gather(pz_v, [s16])
            el2_v[pl.ds(off, 16)] = dx * dx + dy * dy + dz * dz

        pltpu.sync_copy(el2_v, out_hbm.at[pl.ds(base, EPADT)])

    return ek(px, py, pz, src_pad, dst_pad)


def _sc_gather(table, idx, width, chunk):
    n = idx.shape[0]
    per_w = n // NW
    n_ch = per_w // chunk

    @functools.partial(
        pl.kernel,
        out_type=jax.ShapeDtypeStruct((n, width), jnp.float32),
        mesh=_sc_mesh(),
        scratch_types=[
            pltpu.VMEM((chunk,), jnp.int32),
            pltpu.VMEM((chunk, width), jnp.float32),
            pltpu.SemaphoreType.DMA,
        ],
    )
    def gk(table_hbm, idx_hbm, out_hbm, idx_v, rows_v, sem):
        wid = lax.axis_index("s") * NC + lax.axis_index("c")
        base0 = wid * per_w

        @pl.loop(0, n_ch)
        def _(i):
            base = base0 + i * chunk
            pltpu.sync_copy(idx_hbm.at[pl.ds(base, chunk)], idx_v)
            pltpu.async_copy(table_hbm.at[idx_v], rows_v, sem).wait()
            pltpu.sync_copy(rows_v, out_hbm.at[pl.ds(base, chunk)])

    return gk(table, idx)


def _sc_scatter_add(ef, dst, zeros_pad, chunk=200):
    per_t = N_EDGES // NS
    n_ch = per_t // chunk

    @functools.partial(
        pl.kernel,
        out_type=jax.ShapeDtypeStruct((NPAD, D), jnp.float32),
        mesh=_sc_mesh(),
        scratch_types=[
            pltpu.VMEM_SHARED((NPAD, 128), jnp.float32),
            pltpu.VMEM((chunk,), jnp.int32),
            pltpu.VMEM((chunk, 128), jnp.float32),
        ],
    )
    def sk(ef_hbm, dst_hbm, zeros_hbm, out_hbm, accum, idx_v, ef_v):
        c = lax.axis_index("c")
        s = lax.axis_index("s")
        pltpu.sync_copy(zeros_hbm.at[pl.ds(s * RPT, RPT)],
                        accum.at[pl.ds(s * RPT, RPT)])
        plsc.subcore_barrier()

        @pl.loop(0, n_ch)
        def _(i):
            base = s * per_t + i * chunk
            pltpu.sync_copy(dst_hbm.at[pl.ds(base, chunk)], idx_v)
            pltpu.sync_copy(ef_hbm.at[pl.ds(base, chunk), pl.ds(c * 128, 128)],
                            ef_v)
            pltpu.sync_copy(ef_v, accum.at[idx_v], add=True)

        plsc.subcore_barrier()
        pltpu.sync_copy(accum.at[pl.ds(s * RPT, RPT)],
                        out_hbm.at[pl.ds(s * RPT, RPT), pl.ds(c * 128, 128)])

    return sk(ef, dst, zeros_pad)


def _silu(v):
    return v * jax.nn.sigmoid(v)


def _full(shape):
    return pl.BlockSpec(shape, lambda i: (0, 0))


def _rows(blk, width):
    return pl.BlockSpec((blk, width), lambda i: (i, 0))


def _pre_kernel(x, inW, inb, eW1h, eb1, nW, nb):
    def body(x_ref, inW_ref, inb_ref, eW1h_ref, eb1_ref, nW_ref, nb_ref,
             h_ref, hW_ref, hn_ref):
        h = jnp.dot(x_ref[...], inW_ref[...],
                    preferred_element_type=jnp.float32) + inb_ref[...]
        h_ref[...] = h
        hW_ref[...] = jnp.dot(h, eW1h_ref[...],
                              preferred_element_type=jnp.float32) + eb1_ref[...]
        hn_ref[...] = jnp.dot(h, nW_ref[...],
                              preferred_element_type=jnp.float32) + nb_ref[...]

    grid = N_NODES // NODE_BLK
    o = jax.ShapeDtypeStruct((N_NODES, D), jnp.float32)
    return pl.pallas_call(
        body,
        grid=(grid,),
        in_specs=[_rows(NODE_BLK, D), _full((D, D)), _full((1, D)),
                  _full((D, D)), _full((1, D)), _full((D, D)), _full((1, D))],
        out_specs=[_rows(NODE_BLK, D)] * 3,
        out_shape=[o, o, o],
    )(x, inW, inb, eW1h, eb1, nW, nb)


def _edge_kernel(hWsrc, el2c, rW1p, rb1p, rW2p, rb2p, eW1rp, eW2, eb2):
    def body(hWsrc_ref, el2_ref, rW1_ref, rb1_ref, rW2_ref,
             rb2_ref, eW1r_ref, eW2_ref, eb2_ref, out_ref):
        el = jnp.sqrt(el2_ref[...])
        r1 = _silu(el * rW1_ref[...] + rb1_ref[...])
        r2 = _silu(jnp.dot(r1, rW2_ref[...],
                           preferred_element_type=jnp.float32) + rb2_ref[...])
        ef1 = _silu(hWsrc_ref[...]
                    + jnp.dot(r2, eW1r_ref[...],
                              preferred_element_type=jnp.float32))
        ef2 = _silu(jnp.dot(ef1, eW2_ref[...],
                            preferred_element_type=jnp.float32) + eb2_ref[...])
        cf = jnp.clip(1.0 - (el * (1.0 / CUTOFF)) ** 2, 0.0, 1.0)
        cf = cf * (el < CUTOFF).astype(jnp.float32)
        out_ref[...] = ef2 * cf

    grid = N_EDGES // EDGE_BLK
    return pl.pallas_call(
        body,
        grid=(grid,),
        in_specs=[_rows(EDGE_BLK, D), _rows(EDGE_BLK, 1),
                  _full((1, 128)), _full((1, 128)), _full((128, 128)),
                  _full((1, 128)), _full((128, D)), _full((D, D)),
                  _full((1, D))],
        out_specs=_rows(EDGE_BLK, D),
        out_shape=jax.ShapeDtypeStruct((N_EDGES, D), jnp.float32),
    )(hWsrc, el2c, rW1p, rb1p, rW2p, rb2p, eW1rp, eW2, eb2)


def _brow_spec():
    return pl.BlockSpec((1, 1, NODE_BLK), lambda i: (i, 0, 0))


def _update_stats_kernel(h, hn, aggr_pad, batch_row):
    def body(h_ref, hn_ref, aggr_ref, b_ref, h1_ref, st_ref):
        h1 = h_ref[...] + hn_ref[...] + aggr_ref[...]
        h1_ref[...] = h1
        rowsum = jnp.sum(h1, axis=1, keepdims=True)
        rowsq = jnp.sum(h1 * h1, axis=1, keepdims=True)
        ones = jnp.ones_like(rowsum)
        zpad = jnp.zeros((NODE_BLK, 125), jnp.float32)
        mat = jnp.concatenate([rowsum, rowsq, ones, zpad], axis=1)
        iota = lax.broadcasted_iota(jnp.int32, (G, NODE_BLK), 0)
        oh = (b_ref[...].reshape(1, NODE_BLK) == iota).astype(jnp.float32)
        st = jnp.dot(oh, mat, preferred_element_type=jnp.float32)

        @pl.when(pl.program_id(0) == 0)
        def _():
            st_ref[...] = jnp.zeros_like(st_ref)

        st_ref[...] += st

    grid = N_NODES // NODE_BLK
    return pl.pallas_call(
        body,
        grid=(grid,),
        in_specs=[_rows(NODE_BLK, D), _rows(NODE_BLK, D), _rows(NODE_BLK, D),
                  _brow_spec()],
        out_specs=[_rows(NODE_BLK, D), _full((G, 128))],
        out_shape=[jax.ShapeDtypeStruct((N_NODES, D), jnp.float32),
                   jax.ShapeDtypeStruct((G, 128), jnp.float32)],
    )(h, hn, aggr_pad, batch_row)


def _ln_stats_common(st_ref):
    s1 = st_ref[:, 0:1]
    s2 = st_ref[:, 1:2]
    cnt = st_ref[:, 2:3]
    norm = jnp.maximum(cnt, 1.0) * float(D)
    mean = s1 / norm
    var = s2 / norm - mean * mean
    rstd = lax.rsqrt(var + EPS)
    return mean, rstd


def _ln_next_kernel(h1, batchf, stats, w, b, eW1h, eb1, nW, nb):
    def body(h1_ref, b_ref, st_ref, w_ref, bb_ref, eW1h_ref, eb1_ref,
             nW_ref, nb_ref, h_ref, hW_ref, hn_ref):
        mean, rstd = _ln_stats_common(st_ref)
        iota = lax.broadcasted_iota(jnp.int32, (NODE_BLK, G), 1)
        ohn = (b_ref[...] == iota).astype(jnp.float32)
        meanN = jnp.dot(ohn, mean, preferred_element_type=jnp.float32)
        rstdN = jnp.dot(ohn, rstd, preferred_element_type=jnp.float32)
        out = (h1_ref[...] - meanN) * rstdN * w_ref[...] + bb_ref[...]
        out = jnp.maximum(out, 0.0)
        h_ref[...] = out
        hW_ref[...] = jnp.dot(out, eW1h_ref[...],
                              preferred_element_type=jnp.float32) + eb1_ref[...]
        hn_ref[...] = jnp.dot(out, nW_ref[...],
                              preferred_element_type=jnp.float32) + nb_ref[...]

    grid = N_NODES // NODE_BLK
    o = jax.ShapeDtypeStruct((N_NODES, D), jnp.float32)
    return pl.pallas_call(
        body,
        grid=(grid,),
        in_specs=[_rows(NODE_BLK, D), _rows(NODE_BLK, 1), _full((G, 128)),
                  _full((1, D)), _full((1, D)), _full((D, D)), _full((1, D)),
                  _full((D, D)), _full((1, D))],
        out_specs=[_rows(NODE_BLK, D)] * 3,
        out_shape=[o, o, o],
    )(h1, batchf, stats, w, b, eW1h, eb1, nW, nb)


def _ln_pool_kernel(h1, batch_col, batch_row, stats, w, b):
    def body(h1_ref, b_ref, br_ref, st_ref, w_ref, bb_ref, pool_ref):
        mean, rstd = _ln_stats_common(st_ref)
        iota = lax.broadcasted_iota(jnp.int32, (NODE_BLK, G), 1)
        ohn = (b_ref[...] == iota).astype(jnp.float32)
        meanN = jnp.dot(ohn, mean, preferred_element_type=jnp.float32)
        rstdN = jnp.dot(ohn, rstd, preferred_element_type=jnp.float32)
        out = (h1_ref[...] - meanN) * rstdN * w_ref[...] + bb_ref[...]
        iota2 = lax.broadcasted_iota(jnp.int32, (G, NODE_BLK), 0)
        oh = (br_ref[...].reshape(1, NODE_BLK) == iota2).astype(jnp.float32)
        pool = jnp.dot(oh, out, preferred_element_type=jnp.float32)

        @pl.when(pl.program_id(0) == 0)
        def _():
            pool_ref[...] = jnp.zeros_like(pool_ref)

        pool_ref[...] += pool

    grid = N_NODES // NODE_BLK
    return pl.pallas_call(
        body,
        grid=(grid,),
        in_specs=[_rows(NODE_BLK, D), _rows(NODE_BLK, 1), _brow_spec(),
                  _full((G, 128)), _full((1, D)), _full((1, D))],
        out_specs=_full((G, D)),
        out_shape=jax.ShapeDtypeStruct((G, D), jnp.float32),
    )(h1, batch_col, batch_row, stats, w, b)


def _head_kernel(pool, stats, l0W, l0b, l1W, l1b, lW, lb):
    def body(pool_ref, st_ref, l0W_ref, l0b_ref, l1W_ref, l1b_ref,
             lW_ref, lb_ref, out_ref, z_ref):
        cnt = st_ref[:, 2:3]
        g = pool_ref[...] / jnp.maximum(cnt, 1.0)
        g = jnp.maximum(jnp.dot(g, l0W_ref[...],
                                preferred_element_type=jnp.float32)
                        + l0b_ref[...], 0.0)
        z = jnp.maximum(jnp.dot(g, l1W_ref[...],
                                preferred_element_type=jnp.float32)
                        + l1b_ref[...], 0.0)
        z_ref[...] = z
        out_ref[...] = jnp.dot(z, lW_ref[...],
                               preferred_element_type=jnp.float32) + lb_ref[...]

    return pl.pallas_call(
        body,
        grid=(1,),
        in_specs=[_full((G, D)), _full((G, 128)), _full((D, D)),
                  _full((1, D)), _full((D, D)), _full((1, D)),
                  _full((D, 128)), _full((1, 128))],
        out_specs=[_full((G, 128)), _full((G, D))],
        out_shape=[jax.ShapeDtypeStruct((G, 128), jnp.float32),
                   jax.ShapeDtypeStruct((G, D), jnp.float32)],
    )(pool, stats, l0W, l0b, l1W, l1b, lW, lb)


def _pad_conv(cp):
    eW1h = cp["eW1"][:D]
    eW1rp = jnp.zeros((128, D), jnp.float32).at[:4].set(cp["eW1"][D:])
    rW1p = jnp.zeros((1, 128), jnp.float32).at[:, :16].set(cp["rW1"])
    rb1p = jnp.zeros((1, 128), jnp.float32).at[0, :16].set(cp["rb1"])
    rW2p = jnp.zeros((128, 128), jnp.float32).at[:16, :4].set(cp["rW2"])
    rb2p = jnp.zeros((1, 128), jnp.float32).at[0, :4].set(cp["rb2"])
    return dict(eW1h=eW1h, eb1=cp["eb1"].reshape(1, D), eW1rp=eW1rp,
                rW1p=rW1p, rb1p=rb1p, rW2p=rW2p, rb2p=rb2p,
                eW2=cp["eW2"], eb2=cp["eb2"].reshape(1, D),
                nW=cp["nW"], nb=cp["nb"].reshape(1, D))


def kernel(x, edge_index, edge_attr, batch, pos, params):
    p = params
    src = edge_index[0]
    dst = edge_index[1]
    batch_col = batch.reshape(N_NODES, 1)
    batch_row = batch.reshape(N_NODES // NODE_BLK, 1, NODE_BLK)
    zeros_pad = jnp.zeros((NPAD, 128), jnp.float32)
    epad = jnp.zeros((EPAD - N_EDGES,), jnp.int32)
    src_pad = jnp.concatenate([src, epad])
    dst_pad = jnp.concatenate([dst, epad])

    convs = [_pad_conv(p["conv1"]), _pad_conv(p["conv2"]), _pad_conv(p["conv3"])]
    lns = [(p["n1w"].reshape(1, D), p["n1b"].reshape(1, D)),
           (p["n2w"].reshape(1, D), p["n2b"].reshape(1, D)),
           (p["n3w"].reshape(1, D), p["n3b"].reshape(1, D))]

    el2 = _sc_el2(pos[:, 0], pos[:, 1], pos[:, 2], src_pad, dst_pad)
    el2c = el2[:N_EDGES].reshape(N_EDGES, 1)

    c0 = convs[0]
    h, hW, hn = _pre_kernel(x, p["inW"], p["inb"].reshape(1, D),
                            c0["eW1h"], c0["eb1"], c0["nW"], c0["nb"])

    pool = None
    stats = None
    for li in range(3):
        c = convs[li]
        hWsrc = _sc_gather(hW, src, D, 200)
        ef = _edge_kernel(hWsrc, el2c, c["rW1p"], c["rb1p"],
                          c["rW2p"], c["rb2p"], c["eW1rp"], c["eW2"], c["eb2"])
        aggr_pad = _sc_scatter_add(ef, dst, zeros_pad)
        h1, stats = _update_stats_kernel(h, hn, aggr_pad, batch_row)
        w, b = lns[li]
        if li < 2:
            cn = convs[li + 1]
            h, hW, hn = _ln_next_kernel(h1, batch_col, stats, w, b,
                                        cn["eW1h"], cn["eb1"],
                                        cn["nW"], cn["nb"])
        else:
            pool = _ln_pool_kernel(h1, batch_col, batch_row, stats, w, b)

    out, z = _head_kernel(pool, stats, p["l0W"], p["l0b"].reshape(1, D),
                          p["l1W"], p["l1b"].reshape(1, D),
                          p["lW"], p["lb"].reshape(1, 128))
    return (out, z)
